# baseline (speedup 1.0000x reference)
import functools

import jax
import jax.numpy as jnp
from jax import lax
from jax.experimental import pallas as pl
from jax.experimental.pallas import tpu as pltpu
from jax.experimental.pallas import tpu_sc as plsc

N = 16384
K = 256
ROWS, COLS = 360, 180
NC, NS, LANES = 2, 16, 16
NW = NC * NS
BPW = N // NW
CHUNK = 128
NCHUNK = BPW // CHUNK

_mesh = plsc.VectorSubcoreMesh(core_axis_name="c", subcore_axis_name="s")


@functools.partial(
    pl.kernel,
    out_type=jax.ShapeDtypeStruct((N, K), jnp.float32),
    mesh=_mesh,
    scratch_types=[
        pltpu.VMEM((CHUNK,), jnp.float32),
        pltpu.VMEM((CHUNK,), jnp.float32),
        pltpu.VMEM((CHUNK,), jnp.int32),
        pltpu.VMEM((CHUNK,), jnp.int32),
        pltpu.VMEM((CHUNK,), jnp.float32),
        pltpu.VMEM((CHUNK,), jnp.float32),
        pltpu.VMEM((CHUNK, K), jnp.float32),
        pltpu.VMEM((CHUNK, K), jnp.float32),
        pltpu.SemaphoreType.DMA,
        pltpu.SemaphoreType.DMA,
    ],
)
def _sc_lookup(table, lon_in, lat_in, out, lon_v, lat_v, if_v, ic_v, wf_v,
               wc_v, bf, bc, semf, semc):
    wid = lax.axis_index("s") * NC + lax.axis_index("c")
    base = wid * BPW

    for ch in range(NCHUNK):
        cbase = base + ch * CHUNK
        pltpu.sync_copy(lon_in.at[pl.ds(cbase, CHUNK)], lon_v)
        pltpu.sync_copy(lat_in.at[pl.ds(cbase, CHUNK)], lat_v)

        for s in range(CHUNK // LANES):
            sl = pl.ds(s * LANES, LANES)
            r = lon_v[sl] + 180.0
            c = lat_v[sl] + 90.0
            fr = r.astype(jnp.int32)
            fc = c.astype(jnp.int32)
            fa = r - fr.astype(jnp.float32)
            fb = c - fc.astype(jnp.float32)
            cr = jnp.where(fa > 0.0, fr + 1, fr)
            cc = jnp.where(fb > 0.0, fc + 1, fc)
            frc = jnp.minimum(fr, ROWS - 1)
            fcc = jnp.minimum(fc, COLS - 1)
            crc = jnp.minimum(cr, ROWS - 1)
            ccc = jnp.minimum(cc, COLS - 1)
            if_v[sl] = frc * COLS + fcc
            ic_v[sl] = crc * COLS + ccc
            omb = 1.0 - fb
            wf_v[sl] = (1.0 - fa) * omb
            wc_v[sl] = fa * omb

        cpf = pltpu.async_copy(table.at[if_v], bf, semf)
        cpc = pltpu.async_copy(table.at[ic_v], bc, semc)
        cpf.wait()
        cpc.wait()

        def combine(g, carry):
            gbase = g * LANES
            wf16 = wf_v[pl.ds(gbase, LANES)]
            wc16 = wc_v[pl.ds(gbase, LANES)]
            for l in range(LANES):
                wfp = jnp.full((LANES,), wf16[l], jnp.float32)
                wcp = jnp.full((LANES,), wc16[l], jnp.float32)
                p = gbase + l
                for j in range(K // LANES):
                    js = pl.ds(j * LANES, LANES)
                    bf[p, js] = wfp * bf[p, js] + wcp * bc[p, js]
            return carry
        lax.fori_loop(0, CHUNK // LANES, combine, 0)

        pltpu.sync_copy(bf, out.at[pl.ds(cbase, CHUNK)])


def kernel(lonlat, Ys):
    table = jnp.transpose(Ys, (1, 2, 0)).reshape(ROWS * COLS, K)
    return _sc_lookup(table, lonlat[:, 0], lonlat[:, 1])
